# baseline (speedup 1.0000x reference)
import functools

import jax
import jax.numpy as jnp
from jax.experimental import pallas as pl
from jax.experimental.pallas import tpu as pltpu

TOKENS = 16384
DIM = 2048
N_EXPERTS = 16
K = 2
BT = 512


def _router_block(x_ref, w_ref, gates_ref, vals_ref, idx_ref):
    x = x_ref[...]
    w = w_ref[...]
    logits = jax.lax.dot_general(
        x, w, (((1,), (1,)), ((), ())), preferred_element_type=jnp.float32
    )
    m = jnp.max(logits, axis=-1, keepdims=True)
    e = jnp.exp(logits - m)
    s = jnp.sum(e, axis=-1, keepdims=True)
    gates = e / s
    gates_ref[...] = gates
    iota = jax.lax.broadcasted_iota(jnp.int32, gates.shape, 1)
    v1 = jnp.max(gates, axis=-1, keepdims=True)
    i1 = jnp.min(jnp.where(gates == v1, iota, N_EXPERTS), axis=-1, keepdims=True)
    masked = jnp.where(iota == i1, -jnp.inf, gates)
    v2 = jnp.max(masked, axis=-1, keepdims=True)
    i2 = jnp.min(jnp.where(masked == v2, iota, N_EXPERTS), axis=-1, keepdims=True)
    vals_ref[...] = jnp.concatenate([v1, v2], axis=-1)
    idx_ref[...] = jnp.concatenate([i1, i2], axis=-1)


@jax.jit
def kernel(x, W):
    grid = (TOKENS // BT,)
    gates, vals, idx = pl.pallas_call(
        _router_block,
        grid=grid,
        in_specs=[
            pl.BlockSpec((BT, DIM), lambda i: (i, 0)),
            pl.BlockSpec((N_EXPERTS, DIM), lambda i: (0, 0)),
        ],
        out_specs=[
            pl.BlockSpec((BT, N_EXPERTS), lambda i: (i, 0)),
            pl.BlockSpec((BT, K), lambda i: (i, 0)),
            pl.BlockSpec((BT, K), lambda i: (i, 0)),
        ],
        out_shape=[
            jax.ShapeDtypeStruct((TOKENS, N_EXPERTS), jnp.float32),
            jax.ShapeDtypeStruct((TOKENS, K), jnp.float32),
            jax.ShapeDtypeStruct((TOKENS, K), jnp.int32),
        ],
    )(x, W)
    return (gates, vals, idx)


# speedup vs baseline: 1.1801x; 1.1801x over previous
import functools

import jax
import jax.numpy as jnp
from jax.experimental import pallas as pl
from jax.experimental.pallas import tpu as pltpu

TOKENS = 16384
DIM = 2048
N_EXPERTS = 16
K = 2
BT = 2048


def _router_block(x_ref, w_ref, gates_ref, vals_ref, idx_ref):
    x = x_ref[...]
    w = w_ref[...]
    logits = jax.lax.dot_general(
        x, w, (((1,), (1,)), ((), ())), preferred_element_type=jnp.float32
    )
    m = jnp.max(logits, axis=-1, keepdims=True)
    e = jnp.exp(logits - m)
    s = jnp.sum(e, axis=-1, keepdims=True)
    gates = e / s
    gates_ref[...] = gates
    iota = jax.lax.broadcasted_iota(jnp.int32, gates.shape, 1)
    v1 = jnp.max(gates, axis=-1, keepdims=True)
    i1 = jnp.min(jnp.where(gates == v1, iota, N_EXPERTS), axis=-1, keepdims=True)
    masked = jnp.where(iota == i1, -jnp.inf, gates)
    v2 = jnp.max(masked, axis=-1, keepdims=True)
    i2 = jnp.min(jnp.where(masked == v2, iota, N_EXPERTS), axis=-1, keepdims=True)
    vals_ref[...] = jnp.concatenate([v1, v2], axis=-1)
    idx_ref[...] = jnp.concatenate([i1, i2], axis=-1)


@jax.jit
def kernel(x, W):
    grid = (TOKENS // BT,)
    gates, vals, idx = pl.pallas_call(
        _router_block,
        grid=grid,
        in_specs=[
            pl.BlockSpec((BT, DIM), lambda i: (i, 0)),
            pl.BlockSpec((N_EXPERTS, DIM), lambda i: (0, 0)),
        ],
        out_specs=[
            pl.BlockSpec((BT, N_EXPERTS), lambda i: (i, 0)),
            pl.BlockSpec((BT, K), lambda i: (i, 0)),
            pl.BlockSpec((BT, K), lambda i: (i, 0)),
        ],
        out_shape=[
            jax.ShapeDtypeStruct((TOKENS, N_EXPERTS), jnp.float32),
            jax.ShapeDtypeStruct((TOKENS, K), jnp.float32),
            jax.ShapeDtypeStruct((TOKENS, K), jnp.int32),
        ],
    )(x, W)
    return (gates, vals, idx)


# speedup vs baseline: 1.2086x; 1.0241x over previous
import functools

import jax
import jax.numpy as jnp
from jax.experimental import pallas as pl
from jax.experimental.pallas import tpu as pltpu

TOKENS = 16384
DIM = 2048
N_EXPERTS = 16
K = 2
BT = 1024
NBUF = 4
NCHUNK = TOKENS // BT


def _body(x_hbm, w_ref, gates_ref, vals_ref, idx_ref, bufs, sems):
    i = pl.program_id(0)

    def issue(step, slot):
        pltpu.make_async_copy(
            x_hbm.at[pl.ds(step * BT, BT), :], bufs.at[slot], sems.at[slot]
        ).start()

    @pl.when(i == 0)
    def _prime():
        for s in range(min(NBUF - 1, NCHUNK)):
            issue(s, s)

    @pl.when(i + NBUF - 1 < NCHUNK)
    def _ahead():
        issue(i + NBUF - 1, (i + NBUF - 1) % NBUF)

    slot = i % NBUF
    pltpu.make_async_copy(
        x_hbm.at[pl.ds(i * BT, BT), :], bufs.at[slot], sems.at[slot]
    ).wait()

    xb = bufs[slot]
    logits = jax.lax.dot_general(
        xb, w_ref[...], (((1,), (1,)), ((), ())),
        preferred_element_type=jnp.float32,
    )
    m = jnp.max(logits, axis=-1, keepdims=True)
    e = jnp.exp(logits - m)
    s = jnp.sum(e, axis=-1, keepdims=True)
    gates = e / s
    gates_ref[...] = gates
    iota = jax.lax.broadcasted_iota(jnp.int32, gates.shape, 1)
    v1 = jnp.max(gates, axis=-1, keepdims=True)
    i1 = jnp.min(jnp.where(gates == v1, iota, N_EXPERTS), axis=-1, keepdims=True)
    masked = jnp.where(iota == i1, -jnp.inf, gates)
    v2 = jnp.max(masked, axis=-1, keepdims=True)
    i2 = jnp.min(jnp.where(masked == v2, iota, N_EXPERTS), axis=-1, keepdims=True)
    vals_ref[...] = jnp.concatenate([v1, v2], axis=-1)
    idx_ref[...] = jnp.concatenate([i1, i2], axis=-1)


@jax.jit
def kernel(x, W):
    grid = (NCHUNK,)
    gates, vals, idx = pl.pallas_call(
        _body,
        grid=grid,
        in_specs=[
            pl.BlockSpec(memory_space=pltpu.HBM),
            pl.BlockSpec((N_EXPERTS, DIM), lambda i: (0, 0)),
        ],
        out_specs=[
            pl.BlockSpec((BT, N_EXPERTS), lambda i: (i, 0)),
            pl.BlockSpec((BT, K), lambda i: (i, 0)),
            pl.BlockSpec((BT, K), lambda i: (i, 0)),
        ],
        out_shape=[
            jax.ShapeDtypeStruct((TOKENS, N_EXPERTS), jnp.float32),
            jax.ShapeDtypeStruct((TOKENS, K), jnp.float32),
            jax.ShapeDtypeStruct((TOKENS, K), jnp.int32),
        ],
        scratch_shapes=[
            pltpu.VMEM((NBUF, BT, DIM), jnp.float32),
            pltpu.SemaphoreType.DMA((NBUF,)),
        ],
    )(x, W)
    return (gates, vals, idx)
